# baseline (speedup 1.0000x reference)
import functools

import jax
import jax.numpy as jnp
from jax import lax
from jax.experimental import pallas as pl
from jax.experimental.pallas import tpu as pltpu
from jax.experimental.pallas import tpu_sc as plsc

_B, _L, _H = 4, 4096, 512
_N = _B * _L
_NW = 32
_HALF = _N // 2
_TPW = _HALF // _NW
_C = 32
_NCH = _TPW // _C
_PC = 16
_HC = _H // 16
_LANES = 16
_RC = 900
_RC_PT = 64
_CZ = 360
_CZP = 384
_CZB_PT = 96


def _hs(h):
    return pl.ds(h * _LANES, _LANES)


def _main_phase(start, core, wid,
                cid_h, rid_h, col_h, clr_h, rol_h, par_h,
                coord_h, rcd_h, czd_h, out_h,
                cidx, rcidx, czidx, tmpa, tmpb, tmpc,
                cbuf, rcbuf, czbuf,
                sem_g0, sem_g1, sem_o0, sem_o1):
    qbase = start // 128 + wid * (_TPW // 128)
    base = wid * _TPW
    bloc = core + start // _L

    pltpu.sync_copy(cid_h.at[pl.ds(qbase, _TPW // 128)], cidx)

    def idx_chunk(q, carry):
        pltpu.sync_copy(rid_h.at[pl.ds(qbase + q, 1)], tmpa)
        pltpu.sync_copy(col_h.at[pl.ds(qbase + q, 1)], tmpb)
        for j in range(8):
            sl = pl.ds(j * _LANES, _LANES)
            rcidx[q, sl] = tmpa[0, sl] * 30 + tmpb[0, sl]
        pltpu.sync_copy(clr_h.at[pl.ds(qbase + q, 1)], tmpa)
        pltpu.sync_copy(rol_h.at[pl.ds(qbase + q, 1)], tmpb)
        pltpu.sync_copy(par_h.at[pl.ds(qbase + q, 1)], tmpc)
        for j in range(8):
            sl = pl.ds(j * _LANES, _LANES)
            czidx[q, sl] = (tmpa[0, sl] * 36 + tmpb[0, sl] * 9 + tmpc[0, sl]
                            + bloc * _CZP)
        return carry

    lax.fori_loop(0, _TPW // 128, idx_chunk, 0)

    def gathers_on(g, sem):
        s3 = lax.rem(g, 3)
        s2 = lax.rem(g, 2)
        q = g // (128 // _C)
        o = lax.rem(g, 128 // _C) * _C
        return (
            pltpu.make_async_copy(coord_h.at[cidx.at[q, pl.ds(o, _C)]],
                                  cbuf.at[s3], sem),
            pltpu.make_async_copy(
                rcd_h.at[core].at[rcidx.at[q, pl.ds(o, _C)]],
                rcbuf.at[s2], sem),
            pltpu.make_async_copy(
                czd_h.at[core].at[czidx.at[q, pl.ds(o, _C)]],
                czbuf.at[s2], sem),
        )

    def issue(g, sem):
        for cp in gathers_on(g, sem):
            cp.start()

    def wait_gathers(g, sem):
        for cp in gathers_on(g, sem):
            cp.wait()

    def store_cp(g, sem):
        return pltpu.make_async_copy(
            cbuf.at[lax.rem(g, 3)],
            out_h.at[pl.ds(base + g * _C, _C)], sem)

    issue(0, sem_g0)

    def chunk_body(g, carry):
        s3 = lax.rem(g, 3)
        s2 = lax.rem(g, 2)
        even = s2 == 0

        @pl.when(g >= 2)
        def _():
            @pl.when(even)
            def _():
                store_cp(g - 2, sem_o0).wait()

            @pl.when(jnp.logical_not(even))
            def _():
                store_cp(g - 2, sem_o1).wait()

        @pl.when(g + 1 < _NCH)
        def _():
            @pl.when(even)
            def _():
                issue(g + 1, sem_g1)

            @pl.when(jnp.logical_not(even))
            def _():
                issue(g + 1, sem_g0)

        @pl.when(even)
        def _():
            wait_gathers(g, sem_g0)

        @pl.when(jnp.logical_not(even))
        def _():
            wait_gathers(g, sem_g1)

        def token_body(t):
            for h in range(_HC):
                cbuf[s3, t, _hs(h)] = (cbuf[s3, t, _hs(h)]
                                       + rcbuf[s2, t, _hs(h)]
                                       + czbuf[s2, t, _hs(h)])

        plsc.parallel_loop(0, _C, unroll=4)(token_body)

        @pl.when(even)
        def _():
            store_cp(g, sem_o0).start()

        @pl.when(jnp.logical_not(even))
        def _():
            store_cp(g, sem_o1).start()

        return carry

    lax.fori_loop(0, _NCH, chunk_body, 0)
    store_cp(_NCH - 2, sem_o0).wait()
    store_cp(_NCH - 1, sem_o1).wait()


def _sc_body_a(cid_h, rid_h, col_h, clr_h, rol_h, par_h, puz_h,
               coord_h, rowt_h, colt_h, colort_h, rolet_h, pairt_h, puzt_h,
               out_h, rcd_h, czd_h,
               cidx, rcidx, czidx, tmpa, tmpb, tmpc, pzrow,
               cbuf, rcbuf, czbuf,
               sem_g0, sem_g1, sem_o0, sem_o1):
    core = lax.axis_index("c")
    sub = lax.axis_index("s")
    wid = core * 16 + sub
    iota = jnp.arange(_LANES, dtype=jnp.int32)

    pltpu.sync_copy(puz_h.at[pl.ds((sub // 4) * 32, 1)], tmpa)
    pltpu.async_copy(puzt_h.at[tmpa.at[0, pl.ds(0, _LANES)]],
                     cbuf.at[0, pl.ds(0, _PC)], sem_g0).wait()
    for h in range(_HC):
        pzrow[0, _hs(h)] = cbuf[0, 0, _hs(h)]

    def sum2_to_cz0(t):
        for h in range(_HC):
            czbuf[0, t, _hs(h)] = cbuf[0, t, _hs(h)] + cbuf[1, t, _hs(h)]

    def rc_chunk(i, carry):
        st = i * _PC
        kf = jnp.minimum((sub * _RC_PT + st + iota).astype(jnp.float32),
                         float(_RC - 1))
        k = kf.astype(jnp.int32)
        r = ((kf + 0.5) * (1.0 / 30.0)).astype(jnp.int32)
        c = k - r * 30
        tmpa[0, pl.ds(0, _LANES)] = r
        tmpb[0, pl.ds(0, _LANES)] = c
        pltpu.async_copy(rowt_h.at[tmpa.at[0, pl.ds(0, _LANES)]],
                         cbuf.at[0, pl.ds(0, _PC)], sem_g0).wait()
        pltpu.async_copy(colt_h.at[tmpb.at[0, pl.ds(0, _LANES)]],
                         cbuf.at[1, pl.ds(0, _PC)], sem_g0).wait()
        plsc.parallel_loop(0, _PC, unroll=2)(sum2_to_cz0)
        pltpu.sync_copy(czbuf.at[0, pl.ds(0, _PC)],
                        rcd_h.at[core, pl.ds(sub * _RC_PT + st, _PC)])
        return carry

    lax.fori_loop(0, _RC_PT // _PC, rc_chunk, 0)

    def sum4_to_cz0(t):
        for h in range(_HC):
            czbuf[0, t, _hs(h)] = (cbuf[0, t, _hs(h)] + cbuf[1, t, _hs(h)]
                                   + cbuf[2, t, _hs(h)] + pzrow[0, _hs(h)])

    def cz_chunk(i, carry):
        st = i * _PC
        kkf = jnp.minimum(
            (sub * _CZB_PT + st - _CZP * (sub // 4) + iota)
            .astype(jnp.float32), float(_CZ - 1))
        kk = kkf.astype(jnp.int32)
        cl = ((kkf + 0.5) * (1.0 / 36.0)).astype(jnp.int32)
        r36 = kk - cl * 36
        ro = ((r36.astype(jnp.float32) + 0.5) * (1.0 / 9.0)).astype(jnp.int32)
        pa = r36 - ro * 9
        tmpa[0, pl.ds(0, _LANES)] = cl
        tmpb[0, pl.ds(0, _LANES)] = ro
        tmpc[0, pl.ds(0, _LANES)] = pa
        pltpu.async_copy(colort_h.at[tmpa.at[0, pl.ds(0, _LANES)]],
                         cbuf.at[0, pl.ds(0, _PC)], sem_g0).wait()
        pltpu.async_copy(rolet_h.at[tmpb.at[0, pl.ds(0, _LANES)]],
                         cbuf.at[1, pl.ds(0, _PC)], sem_g0).wait()
        pltpu.async_copy(pairt_h.at[tmpc.at[0, pl.ds(0, _LANES)]],
                         cbuf.at[2, pl.ds(0, _PC)], sem_g0).wait()
        plsc.parallel_loop(0, _PC, unroll=2)(sum4_to_cz0)
        pltpu.sync_copy(czbuf.at[0, pl.ds(0, _PC)],
                        czd_h.at[core, pl.ds(sub * _CZB_PT + st, _PC)])
        return carry

    lax.fori_loop(0, _CZB_PT // _PC, cz_chunk, 0)

    plsc.subcore_barrier()

    _main_phase(0, core, wid,
                cid_h, rid_h, col_h, clr_h, rol_h, par_h,
                coord_h, rcd_h, czd_h, out_h,
                cidx, rcidx, czidx, tmpa, tmpb, tmpc,
                cbuf, rcbuf, czbuf,
                sem_g0, sem_g1, sem_o0, sem_o1)


def _sc_body_b(cid_h, rid_h, col_h, clr_h, rol_h, par_h,
               coord_h, rcd_h, czd_h,
               out_h,
               cidx, rcidx, czidx, tmpa, tmpb, tmpc,
               cbuf, rcbuf, czbuf,
               sem_g0, sem_g1, sem_o0, sem_o1):
    core = lax.axis_index("c")
    sub = lax.axis_index("s")
    wid = core * 16 + sub
    _main_phase(_HALF, core, wid,
                cid_h, rid_h, col_h, clr_h, rol_h, par_h,
                coord_h, rcd_h, czd_h, out_h,
                cidx, rcidx, czidx, tmpa, tmpb, tmpc,
                cbuf, rcbuf, czbuf,
                sem_g0, sem_g1, sem_o0, sem_o1)


_IDX_SCRATCH = [
    pltpu.VMEM((_TPW // 128, 128), jnp.int32),
    pltpu.VMEM((_TPW // 128, 128), jnp.int32),
    pltpu.VMEM((_TPW // 128, 128), jnp.int32),
    pltpu.VMEM((1, 128), jnp.int32),
    pltpu.VMEM((1, 128), jnp.int32),
    pltpu.VMEM((1, 128), jnp.int32),
]

_BUF_SCRATCH = [
    pltpu.VMEM((3, _C, _H), jnp.float32),
    pltpu.VMEM((2, _C, _H), jnp.float32),
    pltpu.VMEM((2, _C, _H), jnp.float32),
    pltpu.SemaphoreType.DMA,
    pltpu.SemaphoreType.DMA,
    pltpu.SemaphoreType.DMA,
    pltpu.SemaphoreType.DMA,
]


def _tc_ln_body(x_ref, g_ref, b_ref, o_ref):
    x = x_ref[...]
    m = jnp.mean(x, axis=1, keepdims=True)
    xc = x - m
    v = jnp.mean(xc * xc, axis=1, keepdims=True)
    o_ref[...] = xc * lax.rsqrt(v + 1e-5) * g_ref[...] + b_ref[...]


_TC_BLK = 1024


def _tc_ln(x, g, b):
    return pl.pallas_call(
        _tc_ln_body,
        grid=(_HALF // _TC_BLK,),
        in_specs=[
            pl.BlockSpec((_TC_BLK, _H), lambda i: (i, 0)),
            pl.BlockSpec((_H,), lambda i: (0,)),
            pl.BlockSpec((_H,), lambda i: (0,)),
        ],
        out_specs=pl.BlockSpec((_TC_BLK, _H), lambda i: (i, 0)),
        out_shape=jax.ShapeDtypeStruct((_HALF, _H), jnp.float32),
    )(x, g, b)


@jax.jit
def _launch(cid, rid, col, clr, rol, par, puz,
            coord_table, row_table, col_table, color_table, role_table,
            pair_table, puzzle_table, ln_gamma, ln_beta):
    mesh = plsc.VectorSubcoreMesh(core_axis_name="c", subcore_axis_name="s")
    run_a = pl.kernel(
        _sc_body_a,
        out_type=(
            jax.ShapeDtypeStruct((_HALF, _H), jnp.float32),
            jax.ShapeDtypeStruct((2, 16 * _RC_PT, _H), jnp.float32),
            jax.ShapeDtypeStruct((2, 16 * _CZB_PT, _H), jnp.float32),
        ),
        mesh=mesh,
        scratch_types=_IDX_SCRATCH
        + [pltpu.VMEM((1, _H), jnp.float32)]
        + _BUF_SCRATCH,
    )
    run_b = pl.kernel(
        _sc_body_b,
        out_type=jax.ShapeDtypeStruct((_HALF, _H), jnp.float32),
        mesh=mesh,
        scratch_types=_IDX_SCRATCH + _BUF_SCRATCH,
    )
    x1, rcd, czd = run_a(cid, rid, col, clr, rol, par, puz,
                         coord_table, row_table, col_table, color_table,
                         role_table, pair_table, puzzle_table)
    x2 = run_b(cid, rid, col, clr, rol, par,
               coord_table, rcd, czd)
    y1 = _tc_ln(x1, ln_gamma, ln_beta)
    y2 = _tc_ln(x2, ln_gamma, ln_beta)
    return jnp.concatenate([y1, y2], axis=0)


def kernel(coord_ids, rows, cols, colors, roles, pair_ids, puzzle_id,
           coord_table, row_table, col_table, color_table, role_table,
           pair_table, puzzle_table, ln_gamma, ln_beta):
    def prep(x):
        return x.astype(jnp.int32).reshape(_N // 128, 128)

    puz = jnp.broadcast_to(puzzle_id.astype(jnp.int32)[:, None], (_B, _L))
    y = _launch(prep(coord_ids), prep(rows), prep(cols), prep(colors),
                prep(roles), prep(pair_ids), prep(puz),
                coord_table, row_table, col_table, color_table, role_table,
                pair_table, puzzle_table, ln_gamma, ln_beta)
    return y.reshape(_B, _L, _H)


# speedup vs baseline: 1.3708x; 1.3708x over previous
import functools

import jax
import jax.numpy as jnp
from jax import lax
from jax.experimental import pallas as pl
from jax.experimental.pallas import tpu as pltpu
from jax.experimental.pallas import tpu_sc as plsc

_B, _L, _H = 4, 4096, 512
_N = _B * _L
_NW = 32
_TPW = _N // _NW
_C = 32
_NCH = _TPW // _C
_PC = 16
_HC = _H // 16
_LANES = 16
_RC = 900
_RC_PT = 64
_CZ = 360
_CZP = 384
_CZ_PT = 48


def _hs(h):
    return pl.ds(h * _LANES, _LANES)


def _sc_body(cid_h, rid_h, col_h, clr_h, rol_h, par_h, puz_h,
             coord_h, rowt_h, colt_h, colort_h, rolet_h, pairt_h, puzt_h,
             out_h, rcd_h, czd_h,
             cidx, rcidx, czidx, tmpa, tmpb, tmpc, pzrow,
             cbuf, rcbuf, czbuf,
             sem_g0, sem_g1, sem_o0, sem_o1):
    core = lax.axis_index("c")
    sub = lax.axis_index("s")
    wid = core * 16 + sub
    base = wid * _TPW
    qbase = wid * 4
    iota = jnp.arange(_LANES, dtype=jnp.int32)

    pltpu.sync_copy(puz_h.at[pl.ds(qbase, 1)], tmpa)
    pltpu.async_copy(puzt_h.at[tmpa.at[0, pl.ds(0, _LANES)]],
                     cbuf.at[0, pl.ds(0, _PC)], sem_g0).wait()
    for h in range(_HC):
        pzrow[0, _hs(h)] = cbuf[0, 0, _hs(h)]

    def sum2_to_cz0(t):
        for h in range(_HC):
            czbuf[0, t, _hs(h)] = cbuf[0, t, _hs(h)] + cbuf[1, t, _hs(h)]

    def rc_chunk(i, carry):
        st = i * _PC
        kf = jnp.minimum((sub * _RC_PT + st + iota).astype(jnp.float32),
                         float(_RC - 1))
        k = kf.astype(jnp.int32)
        r = ((kf + 0.5) * (1.0 / 30.0)).astype(jnp.int32)
        c = k - r * 30
        tmpa[0, pl.ds(0, _LANES)] = r
        tmpb[0, pl.ds(0, _LANES)] = c
        cp0 = pltpu.async_copy(rowt_h.at[tmpa.at[0, pl.ds(0, _LANES)]],
                               cbuf.at[0, pl.ds(0, _PC)], sem_g0)
        cp1 = pltpu.async_copy(colt_h.at[tmpb.at[0, pl.ds(0, _LANES)]],
                               cbuf.at[1, pl.ds(0, _PC)], sem_g0)
        cp0.wait()
        cp1.wait()
        plsc.parallel_loop(0, _PC, unroll=2)(sum2_to_cz0)
        pltpu.sync_copy(czbuf.at[0, pl.ds(0, _PC)],
                        rcd_h.at[core, pl.ds(sub * _RC_PT + st, _PC)])
        return carry

    lax.fori_loop(0, _RC_PT // _PC, rc_chunk, 0)

    def sum4_to_cz0(t):
        for h in range(_HC):
            czbuf[0, t, _hs(h)] = (cbuf[0, t, _hs(h)] + cbuf[1, t, _hs(h)]
                                   + cbuf[2, t, _hs(h)] + pzrow[0, _hs(h)])

    def cz_chunk(i, carry):
        st = i * _PC
        kkf = jnp.minimum(
            (sub * _CZ_PT + st - _CZP * (sub // 8) + iota).astype(jnp.float32),
            float(_CZ - 1))
        kk = kkf.astype(jnp.int32)
        cl = ((kkf + 0.5) * (1.0 / 36.0)).astype(jnp.int32)
        r36 = kk - cl * 36
        ro = ((r36.astype(jnp.float32) + 0.5) * (1.0 / 9.0)).astype(jnp.int32)
        pa = r36 - ro * 9
        tmpa[0, pl.ds(0, _LANES)] = cl
        tmpb[0, pl.ds(0, _LANES)] = ro
        tmpc[0, pl.ds(0, _LANES)] = pa
        cp0 = pltpu.async_copy(colort_h.at[tmpa.at[0, pl.ds(0, _LANES)]],
                               cbuf.at[0, pl.ds(0, _PC)], sem_g0)
        cp1 = pltpu.async_copy(rolet_h.at[tmpb.at[0, pl.ds(0, _LANES)]],
                               cbuf.at[1, pl.ds(0, _PC)], sem_g0)
        cp2 = pltpu.async_copy(pairt_h.at[tmpc.at[0, pl.ds(0, _LANES)]],
                               cbuf.at[2, pl.ds(0, _PC)], sem_g0)
        cp0.wait()
        cp1.wait()
        cp2.wait()
        plsc.parallel_loop(0, _PC, unroll=2)(sum4_to_cz0)
        pltpu.sync_copy(czbuf.at[0, pl.ds(0, _PC)],
                        czd_h.at[core, pl.ds(sub * _CZ_PT + st, _PC)])
        return carry

    lax.fori_loop(0, _CZ_PT // _PC, cz_chunk, 0)

    pltpu.sync_copy(cid_h.at[pl.ds(qbase, 4)], cidx)
    bloc = sub // 8

    def idx_chunk(q, carry):
        pltpu.sync_copy(rid_h.at[pl.ds(qbase + q, 1)], tmpa)
        pltpu.sync_copy(col_h.at[pl.ds(qbase + q, 1)], tmpb)
        for j in range(8):
            sl = pl.ds(j * _LANES, _LANES)
            rcidx[q, sl] = tmpa[0, sl] * 30 + tmpb[0, sl]
        pltpu.sync_copy(clr_h.at[pl.ds(qbase + q, 1)], tmpa)
        pltpu.sync_copy(rol_h.at[pl.ds(qbase + q, 1)], tmpb)
        pltpu.sync_copy(par_h.at[pl.ds(qbase + q, 1)], tmpc)
        for j in range(8):
            sl = pl.ds(j * _LANES, _LANES)
            czidx[q, sl] = (tmpa[0, sl] * 36 + tmpb[0, sl] * 9 + tmpc[0, sl]
                            + bloc * _CZP)
        return carry

    lax.fori_loop(0, 4, idx_chunk, 0)

    plsc.subcore_barrier()

    def gathers_on(g, sem):
        s3 = lax.rem(g, 3)
        s2 = lax.rem(g, 2)
        q = g // (128 // _C)
        o = lax.rem(g, 128 // _C) * _C
        return (
            pltpu.make_async_copy(coord_h.at[cidx.at[q, pl.ds(o, _C)]],
                                  cbuf.at[s3], sem),
            pltpu.make_async_copy(
                rcd_h.at[core].at[rcidx.at[q, pl.ds(o, _C)]],
                rcbuf.at[s2], sem),
            pltpu.make_async_copy(
                czd_h.at[core].at[czidx.at[q, pl.ds(o, _C)]],
                czbuf.at[s2], sem),
        )

    def issue(g, sem):
        for cp in gathers_on(g, sem):
            cp.start()

    def wait_gathers(g, sem):
        for cp in gathers_on(g, sem):
            cp.wait()

    def store_cp(g, sem):
        return pltpu.make_async_copy(
            cbuf.at[lax.rem(g, 3)],
            out_h.at[pl.ds(base + g * _C, _C)], sem)

    issue(0, sem_g0)

    def chunk_body(g, carry):
        s3 = lax.rem(g, 3)
        s2 = lax.rem(g, 2)
        even = s2 == 0

        @pl.when(g >= 2)
        def _():
            @pl.when(even)
            def _():
                store_cp(g - 2, sem_o0).wait()

            @pl.when(jnp.logical_not(even))
            def _():
                store_cp(g - 2, sem_o1).wait()

        @pl.when(g + 1 < _NCH)
        def _():
            @pl.when(even)
            def _():
                issue(g + 1, sem_g1)

            @pl.when(jnp.logical_not(even))
            def _():
                issue(g + 1, sem_g0)

        @pl.when(even)
        def _():
            wait_gathers(g, sem_g0)

        @pl.when(jnp.logical_not(even))
        def _():
            wait_gathers(g, sem_g1)

        def token_body(t):
            for h in range(_HC):
                cbuf[s3, t, _hs(h)] = (cbuf[s3, t, _hs(h)]
                                       + rcbuf[s2, t, _hs(h)]
                                       + czbuf[s2, t, _hs(h)])

        plsc.parallel_loop(0, _C, unroll=4)(token_body)

        @pl.when(even)
        def _():
            store_cp(g, sem_o0).start()

        @pl.when(jnp.logical_not(even))
        def _():
            store_cp(g, sem_o1).start()

        return carry

    lax.fori_loop(0, _NCH, chunk_body, 0)
    store_cp(_NCH - 2, sem_o0).wait()
    store_cp(_NCH - 1, sem_o1).wait()


def _tc_ln_body(x_ref, g_ref, b_ref, o_ref):
    x = x_ref[...]
    m = jnp.mean(x, axis=1, keepdims=True)
    xc = x - m
    v = jnp.mean(xc * xc, axis=1, keepdims=True)
    o_ref[...] = xc * lax.rsqrt(v + 1e-5) * g_ref[...] + b_ref[...]


_TC_BLK = 1024


def _tc_ln(x, g, b):
    return pl.pallas_call(
        _tc_ln_body,
        grid=(_N // _TC_BLK,),
        in_specs=[
            pl.BlockSpec((_TC_BLK, _H), lambda i: (i, 0)),
            pl.BlockSpec((_H,), lambda i: (0,)),
            pl.BlockSpec((_H,), lambda i: (0,)),
        ],
        out_specs=pl.BlockSpec((_TC_BLK, _H), lambda i: (i, 0)),
        out_shape=jax.ShapeDtypeStruct((_N, _H), jnp.float32),
    )(x, g, b)


@jax.jit
def _launch(cid, rid, col, clr, rol, par, puz,
            coord_table, row_table, col_table, color_table, role_table,
            pair_table, puzzle_table, ln_gamma, ln_beta):
    mesh = plsc.VectorSubcoreMesh(core_axis_name="c", subcore_axis_name="s")
    run = pl.kernel(
        _sc_body,
        out_type=(
            jax.ShapeDtypeStruct((_N, _H), jnp.float32),
            jax.ShapeDtypeStruct((2, 16 * _RC_PT, _H), jnp.float32),
            jax.ShapeDtypeStruct((2, 2 * _CZP, _H), jnp.float32),
        ),
        mesh=mesh,
        scratch_types=[
            pltpu.VMEM((4, 128), jnp.int32),
            pltpu.VMEM((4, 128), jnp.int32),
            pltpu.VMEM((4, 128), jnp.int32),
            pltpu.VMEM((1, 128), jnp.int32),
            pltpu.VMEM((1, 128), jnp.int32),
            pltpu.VMEM((1, 128), jnp.int32),
            pltpu.VMEM((1, _H), jnp.float32),
            pltpu.VMEM((3, _C, _H), jnp.float32),
            pltpu.VMEM((2, _C, _H), jnp.float32),
            pltpu.VMEM((2, _C, _H), jnp.float32),
            pltpu.SemaphoreType.DMA,
            pltpu.SemaphoreType.DMA,
            pltpu.SemaphoreType.DMA,
            pltpu.SemaphoreType.DMA,
        ],
    )
    x, _rcd, _czd = run(cid, rid, col, clr, rol, par, puz,
                        coord_table, row_table, col_table, color_table,
                        role_table, pair_table, puzzle_table)
    return _tc_ln(x, ln_gamma, ln_beta)


def kernel(coord_ids, rows, cols, colors, roles, pair_ids, puzzle_id,
           coord_table, row_table, col_table, color_table, role_table,
           pair_table, puzzle_table, ln_gamma, ln_beta):
    def prep(x):
        return x.astype(jnp.int32).reshape(_N // 128, 128)

    puz = jnp.broadcast_to(puzzle_id.astype(jnp.int32)[:, None], (_B, _L))
    y = _launch(prep(coord_ids), prep(rows), prep(cols), prep(colors),
                prep(roles), prep(pair_ids), prep(puz),
                coord_table, row_table, col_table, color_table, role_table,
                pair_table, puzzle_table, ln_gamma, ln_beta)
    return y.reshape(_B, _L, _H)


# speedup vs baseline: 1.3917x; 1.0152x over previous
import functools

import jax
import jax.numpy as jnp
from jax import lax
from jax.experimental import pallas as pl
from jax.experimental.pallas import tpu as pltpu
from jax.experimental.pallas import tpu_sc as plsc

_B, _L, _H = 4, 4096, 512
_N = _B * _L
_NW = 32
_TPW = _N // _NW
_C = 32
_NCH = _TPW // _C
_PC = 16
_HC = _H // 16
_LANES = 16
_RC = 900
_RC_PT = 64
_CZ = 360
_CZP = 384
_CZ_PT = 48


def _hs(h):
    return pl.ds(h * _LANES, _LANES)


def _sc_body(cid_h, rid_h, col_h, clr_h, rol_h, par_h, puz_h,
             coord_h, rowt_h, colt_h, colort_h, rolet_h, pairt_h, puzt_h,
             out_h, rcd_h, czd_h,
             cidx, rcidx, czidx, tmpa, tmpb, tmpc, pzrow,
             cbuf, rcbuf, czbuf,
             sem_g0, sem_g1, sem_o0, sem_o1):
    core = lax.axis_index("c")
    sub = lax.axis_index("s")
    wid = core * 16 + sub
    base = wid * _TPW
    qbase = wid * 4
    iota = jnp.arange(_LANES, dtype=jnp.int32)

    pltpu.sync_copy(puz_h.at[pl.ds(qbase, 1)], tmpa)
    pltpu.async_copy(puzt_h.at[tmpa.at[0, pl.ds(0, _LANES)]],
                     cbuf.at[0, pl.ds(0, _PC)], sem_g0).wait()
    for h in range(_HC):
        pzrow[0, _hs(h)] = cbuf[0, 0, _hs(h)]

    def sum2_to_cz0(t):
        for h in range(_HC):
            czbuf[0, t, _hs(h)] = cbuf[0, t, _hs(h)] + cbuf[1, t, _hs(h)]

    def rc_chunk(i, carry):
        st = i * _PC
        kf = jnp.minimum((sub * _RC_PT + st + iota).astype(jnp.float32),
                         float(_RC - 1))
        k = kf.astype(jnp.int32)
        r = ((kf + 0.5) * (1.0 / 30.0)).astype(jnp.int32)
        c = k - r * 30
        tmpa[0, pl.ds(0, _LANES)] = r
        tmpb[0, pl.ds(0, _LANES)] = c
        cp0 = pltpu.async_copy(rowt_h.at[tmpa.at[0, pl.ds(0, _LANES)]],
                               cbuf.at[0, pl.ds(0, _PC)], sem_g0)
        cp1 = pltpu.async_copy(colt_h.at[tmpb.at[0, pl.ds(0, _LANES)]],
                               cbuf.at[1, pl.ds(0, _PC)], sem_g0)
        cp0.wait()
        cp1.wait()
        plsc.parallel_loop(0, _PC, unroll=2)(sum2_to_cz0)
        pltpu.sync_copy(czbuf.at[0, pl.ds(0, _PC)],
                        rcd_h.at[core, pl.ds(sub * _RC_PT + st, _PC)])
        return carry

    lax.fori_loop(0, _RC_PT // _PC, rc_chunk, 0)

    def sum4_to_cz0(t):
        for h in range(_HC):
            czbuf[0, t, _hs(h)] = (cbuf[0, t, _hs(h)] + cbuf[1, t, _hs(h)]
                                   + cbuf[2, t, _hs(h)] + pzrow[0, _hs(h)])

    def cz_chunk(i, carry):
        st = i * _PC
        kkf = jnp.minimum(
            (sub * _CZ_PT + st - _CZP * (sub // 8) + iota).astype(jnp.float32),
            float(_CZ - 1))
        kk = kkf.astype(jnp.int32)
        cl = ((kkf + 0.5) * (1.0 / 36.0)).astype(jnp.int32)
        r36 = kk - cl * 36
        ro = ((r36.astype(jnp.float32) + 0.5) * (1.0 / 9.0)).astype(jnp.int32)
        pa = r36 - ro * 9
        tmpa[0, pl.ds(0, _LANES)] = cl
        tmpb[0, pl.ds(0, _LANES)] = ro
        tmpc[0, pl.ds(0, _LANES)] = pa
        cp0 = pltpu.async_copy(colort_h.at[tmpa.at[0, pl.ds(0, _LANES)]],
                               cbuf.at[0, pl.ds(0, _PC)], sem_g0)
        cp1 = pltpu.async_copy(rolet_h.at[tmpb.at[0, pl.ds(0, _LANES)]],
                               cbuf.at[1, pl.ds(0, _PC)], sem_g0)
        cp2 = pltpu.async_copy(pairt_h.at[tmpc.at[0, pl.ds(0, _LANES)]],
                               cbuf.at[2, pl.ds(0, _PC)], sem_g0)
        cp0.wait()
        cp1.wait()
        cp2.wait()
        plsc.parallel_loop(0, _PC, unroll=2)(sum4_to_cz0)
        pltpu.sync_copy(czbuf.at[0, pl.ds(0, _PC)],
                        czd_h.at[core, pl.ds(sub * _CZ_PT + st, _PC)])
        return carry

    lax.fori_loop(0, _CZ_PT // _PC, cz_chunk, 0)

    pltpu.sync_copy(cid_h.at[pl.ds(qbase, 4)], cidx)
    bloc = sub // 8

    def idx_chunk(q, carry):
        pltpu.sync_copy(rid_h.at[pl.ds(qbase + q, 1)], tmpa)
        pltpu.sync_copy(col_h.at[pl.ds(qbase + q, 1)], tmpb)
        for j in range(8):
            sl = pl.ds(j * _LANES, _LANES)
            rcidx[q, sl] = tmpa[0, sl] * 30 + tmpb[0, sl]
        pltpu.sync_copy(clr_h.at[pl.ds(qbase + q, 1)], tmpa)
        pltpu.sync_copy(rol_h.at[pl.ds(qbase + q, 1)], tmpb)
        pltpu.sync_copy(par_h.at[pl.ds(qbase + q, 1)], tmpc)
        for j in range(8):
            sl = pl.ds(j * _LANES, _LANES)
            czidx[q, sl] = (tmpa[0, sl] * 36 + tmpb[0, sl] * 9 + tmpc[0, sl]
                            + bloc * _CZP)
        return carry

    lax.fori_loop(0, 4, idx_chunk, 0)

    plsc.subcore_barrier()

    def gathers_on(g, sem):
        s3 = lax.rem(g, 3)
        s2 = lax.rem(g, 2)
        q = g // (128 // _C)
        o = lax.rem(g, 128 // _C) * _C
        return (
            pltpu.make_async_copy(coord_h.at[cidx.at[q, pl.ds(o, _C)]],
                                  cbuf.at[s3], sem),
            pltpu.make_async_copy(
                rcd_h.at[core].at[rcidx.at[q, pl.ds(o, _C)]],
                rcbuf.at[s2], sem),
            pltpu.make_async_copy(
                czd_h.at[core].at[czidx.at[q, pl.ds(o, _C)]],
                czbuf.at[s2], sem),
        )

    def issue(g, sem):
        for cp in gathers_on(g, sem):
            cp.start()

    def wait_gathers(g, sem):
        for cp in gathers_on(g, sem):
            cp.wait()

    def store_cp(g, sem):
        return pltpu.make_async_copy(
            cbuf.at[lax.rem(g, 3)],
            out_h.at[pl.ds(base + g * _C, _C)], sem)

    issue(0, sem_g0)

    def chunk_body(g, carry):
        s3 = lax.rem(g, 3)
        s2 = lax.rem(g, 2)
        even = s2 == 0

        @pl.when(g >= 2)
        def _():
            @pl.when(even)
            def _():
                store_cp(g - 2, sem_o0).wait()

            @pl.when(jnp.logical_not(even))
            def _():
                store_cp(g - 2, sem_o1).wait()

        @pl.when(g + 1 < _NCH)
        def _():
            @pl.when(even)
            def _():
                issue(g + 1, sem_g1)

            @pl.when(jnp.logical_not(even))
            def _():
                issue(g + 1, sem_g0)

        @pl.when(even)
        def _():
            wait_gathers(g, sem_g0)

        @pl.when(jnp.logical_not(even))
        def _():
            wait_gathers(g, sem_g1)

        def token_body(t):
            for h in range(_HC):
                cbuf[s3, t, _hs(h)] = (cbuf[s3, t, _hs(h)]
                                       + rcbuf[s2, t, _hs(h)]
                                       + czbuf[s2, t, _hs(h)])

        plsc.parallel_loop(0, _C, unroll=4)(token_body)

        @pl.when(even)
        def _():
            store_cp(g, sem_o0).start()

        @pl.when(jnp.logical_not(even))
        def _():
            store_cp(g, sem_o1).start()

        return carry

    lax.fori_loop(0, _NCH, chunk_body, 0)
    store_cp(_NCH - 2, sem_o0).wait()
    store_cp(_NCH - 1, sem_o1).wait()


def _tc_ln_body(x_ref, g_ref, b_ref, o_ref):
    x = x_ref[...]
    m = jnp.mean(x, axis=1, keepdims=True)
    xc = x - m
    v = jnp.mean(xc * xc, axis=1, keepdims=True)
    o_ref[...] = xc * lax.rsqrt(v + 1e-5) * g_ref[...] + b_ref[...]


_TC_BLK = 2048


def _tc_ln(x, g, b):
    return pl.pallas_call(
        _tc_ln_body,
        grid=(_N // _TC_BLK,),
        in_specs=[
            pl.BlockSpec((_TC_BLK, _H), lambda i: (i, 0)),
            pl.BlockSpec((_H,), lambda i: (0,)),
            pl.BlockSpec((_H,), lambda i: (0,)),
        ],
        out_specs=pl.BlockSpec((_TC_BLK, _H), lambda i: (i, 0)),
        out_shape=jax.ShapeDtypeStruct((_N, _H), jnp.float32),
    )(x, g, b)


@jax.jit
def _launch(cid, rid, col, clr, rol, par, puz,
            coord_table, row_table, col_table, color_table, role_table,
            pair_table, puzzle_table, ln_gamma, ln_beta):
    mesh = plsc.VectorSubcoreMesh(core_axis_name="c", subcore_axis_name="s")
    run = pl.kernel(
        _sc_body,
        out_type=(
            jax.ShapeDtypeStruct((_N, _H), jnp.float32),
            jax.ShapeDtypeStruct((2, 16 * _RC_PT, _H), jnp.float32),
            jax.ShapeDtypeStruct((2, 2 * _CZP, _H), jnp.float32),
        ),
        mesh=mesh,
        scratch_types=[
            pltpu.VMEM((4, 128), jnp.int32),
            pltpu.VMEM((4, 128), jnp.int32),
            pltpu.VMEM((4, 128), jnp.int32),
            pltpu.VMEM((1, 128), jnp.int32),
            pltpu.VMEM((1, 128), jnp.int32),
            pltpu.VMEM((1, 128), jnp.int32),
            pltpu.VMEM((1, _H), jnp.float32),
            pltpu.VMEM((3, _C, _H), jnp.float32),
            pltpu.VMEM((2, _C, _H), jnp.float32),
            pltpu.VMEM((2, _C, _H), jnp.float32),
            pltpu.SemaphoreType.DMA,
            pltpu.SemaphoreType.DMA,
            pltpu.SemaphoreType.DMA,
            pltpu.SemaphoreType.DMA,
        ],
    )
    x, _rcd, _czd = run(cid, rid, col, clr, rol, par, puz,
                        coord_table, row_table, col_table, color_table,
                        role_table, pair_table, puzzle_table)
    return _tc_ln(x, ln_gamma, ln_beta)


def kernel(coord_ids, rows, cols, colors, roles, pair_ids, puzzle_id,
           coord_table, row_table, col_table, color_table, role_table,
           pair_table, puzzle_table, ln_gamma, ln_beta):
    def prep(x):
        return x.astype(jnp.int32).reshape(_N // 128, 128)

    puz = jnp.broadcast_to(puzzle_id.astype(jnp.int32)[:, None], (_B, _L))
    y = _launch(prep(coord_ids), prep(rows), prep(cols), prep(colors),
                prep(roles), prep(pair_ids), prep(puz),
                coord_table, row_table, col_table, color_table, role_table,
                pair_table, puzzle_table, ln_gamma, ln_beta)
    return y.reshape(_B, _L, _H)


# speedup vs baseline: 1.4539x; 1.0447x over previous
import functools

import jax
import jax.numpy as jnp
from jax import lax
from jax.experimental import pallas as pl
from jax.experimental.pallas import tpu as pltpu
from jax.experimental.pallas import tpu_sc as plsc

_B, _L, _H = 4, 4096, 512
_N = _B * _L
_NW = 32
_TPW = _N // _NW
_C = 32
_NCH = _TPW // _C
_PC = 16
_HC = _H // 16
_LANES = 16
_RC = 900
_RC_PT = 64
_CZ = 360
_CZP = 384
_CZ_PT = 48


def _hs(h):
    return pl.ds(h * _LANES, _LANES)


def _sc_body(cid_h, rid_h, col_h, clr_h, rol_h, par_h, puz_h,
             coord_h, rowt_h, colt_h, colort_h, rolet_h, pairt_h, puzt_h,
             out_h, rcd_h, czd_h,
             cidx, rcidx, czidx, tmpa, tmpb, tmpc, pzrow,
             cbuf, rcbuf, czbuf,
             sem_g0, sem_g1, sem_o0, sem_o1):
    core = lax.axis_index("c")
    sub = lax.axis_index("s")
    wid = core * 16 + sub
    base = wid * _TPW
    qbase = wid * 4
    iota = jnp.arange(_LANES, dtype=jnp.int32)

    pltpu.sync_copy(puz_h.at[pl.ds(qbase, 1)], tmpa)
    pltpu.async_copy(puzt_h.at[tmpa.at[0, pl.ds(0, _LANES)]],
                     cbuf.at[0, pl.ds(0, _PC)], sem_g0).wait()
    for h in range(_HC):
        pzrow[0, _hs(h)] = cbuf[0, 0, _hs(h)]

    def sum2_to_cz0(t):
        for h in range(_HC):
            czbuf[0, t, _hs(h)] = cbuf[0, t, _hs(h)] + cbuf[1, t, _hs(h)]

    def rc_chunk(i, carry):
        st = i * _PC
        kf = jnp.minimum((sub * _RC_PT + st + iota).astype(jnp.float32),
                         float(_RC - 1))
        k = kf.astype(jnp.int32)
        r = ((kf + 0.5) * (1.0 / 30.0)).astype(jnp.int32)
        c = k - r * 30
        tmpa[0, pl.ds(0, _LANES)] = r
        tmpb[0, pl.ds(0, _LANES)] = c
        cp0 = pltpu.async_copy(rowt_h.at[tmpa.at[0, pl.ds(0, _LANES)]],
                               cbuf.at[0, pl.ds(0, _PC)], sem_g0)
        cp1 = pltpu.async_copy(colt_h.at[tmpb.at[0, pl.ds(0, _LANES)]],
                               cbuf.at[1, pl.ds(0, _PC)], sem_g0)
        cp0.wait()
        cp1.wait()
        plsc.parallel_loop(0, _PC, unroll=2)(sum2_to_cz0)
        pltpu.sync_copy(czbuf.at[0, pl.ds(0, _PC)],
                        rcd_h.at[core, pl.ds(sub * _RC_PT + st, _PC)])
        return carry

    lax.fori_loop(0, _RC_PT // _PC, rc_chunk, 0)

    def sum4_to_cz0(t):
        for h in range(_HC):
            czbuf[0, t, _hs(h)] = (cbuf[0, t, _hs(h)] + cbuf[1, t, _hs(h)]
                                   + cbuf[2, t, _hs(h)] + pzrow[0, _hs(h)])

    def cz_chunk(i, carry):
        st = i * _PC
        kkf = jnp.minimum(
            (sub * _CZ_PT + st - _CZP * (sub // 8) + iota).astype(jnp.float32),
            float(_CZ - 1))
        kk = kkf.astype(jnp.int32)
        cl = ((kkf + 0.5) * (1.0 / 36.0)).astype(jnp.int32)
        r36 = kk - cl * 36
        ro = ((r36.astype(jnp.float32) + 0.5) * (1.0 / 9.0)).astype(jnp.int32)
        pa = r36 - ro * 9
        tmpa[0, pl.ds(0, _LANES)] = cl
        tmpb[0, pl.ds(0, _LANES)] = ro
        tmpc[0, pl.ds(0, _LANES)] = pa
        cp0 = pltpu.async_copy(colort_h.at[tmpa.at[0, pl.ds(0, _LANES)]],
                               cbuf.at[0, pl.ds(0, _PC)], sem_g0)
        cp1 = pltpu.async_copy(rolet_h.at[tmpb.at[0, pl.ds(0, _LANES)]],
                               cbuf.at[1, pl.ds(0, _PC)], sem_g0)
        cp2 = pltpu.async_copy(pairt_h.at[tmpc.at[0, pl.ds(0, _LANES)]],
                               cbuf.at[2, pl.ds(0, _PC)], sem_g0)
        cp0.wait()
        cp1.wait()
        cp2.wait()
        plsc.parallel_loop(0, _PC, unroll=2)(sum4_to_cz0)
        pltpu.sync_copy(czbuf.at[0, pl.ds(0, _PC)],
                        czd_h.at[core, pl.ds(sub * _CZ_PT + st, _PC)])
        return carry

    lax.fori_loop(0, _CZ_PT // _PC, cz_chunk, 0)

    pltpu.sync_copy(cid_h.at[pl.ds(qbase, 4)], cidx)
    bloc = sub // 8

    def idx_chunk(q, carry):
        pltpu.sync_copy(rid_h.at[pl.ds(qbase + q, 1)], tmpa)
        pltpu.sync_copy(col_h.at[pl.ds(qbase + q, 1)], tmpb)
        for j in range(8):
            sl = pl.ds(j * _LANES, _LANES)
            rcidx[q, sl] = tmpa[0, sl] * 30 + tmpb[0, sl]
        pltpu.sync_copy(clr_h.at[pl.ds(qbase + q, 1)], tmpa)
        pltpu.sync_copy(rol_h.at[pl.ds(qbase + q, 1)], tmpb)
        pltpu.sync_copy(par_h.at[pl.ds(qbase + q, 1)], tmpc)
        for j in range(8):
            sl = pl.ds(j * _LANES, _LANES)
            czidx[q, sl] = (tmpa[0, sl] * 36 + tmpb[0, sl] * 9 + tmpc[0, sl]
                            + bloc * _CZP)
        return carry

    lax.fori_loop(0, 4, idx_chunk, 0)

    plsc.subcore_barrier()

    def gathers_on(g, sem):
        s3 = lax.rem(g, 3)
        s2 = lax.rem(g, 2)
        q = g // (128 // _C)
        o = lax.rem(g, 128 // _C) * _C
        return (
            pltpu.make_async_copy(coord_h.at[cidx.at[q, pl.ds(o, _C)]],
                                  cbuf.at[s3], sem),
            pltpu.make_async_copy(
                rcd_h.at[core].at[rcidx.at[q, pl.ds(o, _C)]],
                rcbuf.at[s2], sem),
            pltpu.make_async_copy(
                czd_h.at[core].at[czidx.at[q, pl.ds(o, _C)]],
                czbuf.at[s2], sem),
        )

    def issue(g, sem):
        for cp in gathers_on(g, sem):
            cp.start()

    def wait_gathers(g, sem):
        for cp in gathers_on(g, sem):
            cp.wait()

    def store_cp(g, sem):
        return pltpu.make_async_copy(
            cbuf.at[lax.rem(g, 3)],
            out_h.at[pl.ds(base + g * _C, _C)], sem)

    issue(0, sem_g0)

    def chunk_body(g, carry):
        s3 = lax.rem(g, 3)
        s2 = lax.rem(g, 2)
        even = s2 == 0

        @pl.when(g >= 2)
        def _():
            @pl.when(even)
            def _():
                store_cp(g - 2, sem_o0).wait()

            @pl.when(jnp.logical_not(even))
            def _():
                store_cp(g - 2, sem_o1).wait()

        @pl.when(g + 1 < _NCH)
        def _():
            @pl.when(even)
            def _():
                issue(g + 1, sem_g1)

            @pl.when(jnp.logical_not(even))
            def _():
                issue(g + 1, sem_g0)

        @pl.when(even)
        def _():
            wait_gathers(g, sem_g0)

        @pl.when(jnp.logical_not(even))
        def _():
            wait_gathers(g, sem_g1)

        def token_body(t):
            for h in range(_HC):
                cbuf[s3, t, _hs(h)] = (cbuf[s3, t, _hs(h)]
                                       + rcbuf[s2, t, _hs(h)]
                                       + czbuf[s2, t, _hs(h)])

        plsc.parallel_loop(0, _C, unroll=4)(token_body)

        @pl.when(even)
        def _():
            store_cp(g, sem_o0).start()

        @pl.when(jnp.logical_not(even))
        def _():
            store_cp(g, sem_o1).start()

        return carry

    lax.fori_loop(0, _NCH, chunk_body, 0)
    store_cp(_NCH - 2, sem_o0).wait()
    store_cp(_NCH - 1, sem_o1).wait()


def _tc_ln_body(x_ref, g_ref, b_ref, o_ref):
    x = x_ref[...]
    m = jnp.mean(x, axis=1, keepdims=True)
    xc = x - m
    v = jnp.mean(xc * xc, axis=1, keepdims=True)
    o_ref[...] = xc * lax.rsqrt(v + 1e-5) * g_ref[...] + b_ref[...]


_TC_BLK = 4096


def _tc_ln(x, g, b):
    return pl.pallas_call(
        _tc_ln_body,
        grid=(_N // _TC_BLK,),
        in_specs=[
            pl.BlockSpec((_TC_BLK, _H), lambda i: (i, 0)),
            pl.BlockSpec((_H,), lambda i: (0,)),
            pl.BlockSpec((_H,), lambda i: (0,)),
        ],
        out_specs=pl.BlockSpec((_TC_BLK, _H), lambda i: (i, 0)),
        out_shape=jax.ShapeDtypeStruct((_N, _H), jnp.float32),
    )(x, g, b)


@jax.jit
def _launch(cid, rid, col, clr, rol, par, puz,
            coord_table, row_table, col_table, color_table, role_table,
            pair_table, puzzle_table, ln_gamma, ln_beta):
    mesh = plsc.VectorSubcoreMesh(core_axis_name="c", subcore_axis_name="s")
    run = pl.kernel(
        _sc_body,
        out_type=(
            jax.ShapeDtypeStruct((_N, _H), jnp.float32),
            jax.ShapeDtypeStruct((2, 16 * _RC_PT, _H), jnp.float32),
            jax.ShapeDtypeStruct((2, 2 * _CZP, _H), jnp.float32),
        ),
        mesh=mesh,
        scratch_types=[
            pltpu.VMEM((4, 128), jnp.int32),
            pltpu.VMEM((4, 128), jnp.int32),
            pltpu.VMEM((4, 128), jnp.int32),
            pltpu.VMEM((1, 128), jnp.int32),
            pltpu.VMEM((1, 128), jnp.int32),
            pltpu.VMEM((1, 128), jnp.int32),
            pltpu.VMEM((1, _H), jnp.float32),
            pltpu.VMEM((3, _C, _H), jnp.float32),
            pltpu.VMEM((2, _C, _H), jnp.float32),
            pltpu.VMEM((2, _C, _H), jnp.float32),
            pltpu.SemaphoreType.DMA,
            pltpu.SemaphoreType.DMA,
            pltpu.SemaphoreType.DMA,
            pltpu.SemaphoreType.DMA,
        ],
    )
    x, _rcd, _czd = run(cid, rid, col, clr, rol, par, puz,
                        coord_table, row_table, col_table, color_table,
                        role_table, pair_table, puzzle_table)
    return _tc_ln(x, ln_gamma, ln_beta)


def kernel(coord_ids, rows, cols, colors, roles, pair_ids, puzzle_id,
           coord_table, row_table, col_table, color_table, role_table,
           pair_table, puzzle_table, ln_gamma, ln_beta):
    def prep(x):
        return x.astype(jnp.int32).reshape(_N // 128, 128)

    puz = jnp.broadcast_to(puzzle_id.astype(jnp.int32)[:, None], (_B, _L))
    y = _launch(prep(coord_ids), prep(rows), prep(cols), prep(colors),
                prep(roles), prep(pair_ids), prep(puz),
                coord_table, row_table, col_table, color_table, role_table,
                pair_table, puzzle_table, ln_gamma, ln_beta)
    return y.reshape(_B, _L, _H)
